# baseline (speedup 1.0000x reference)
import functools

import jax
import jax.numpy as jnp
from jax.experimental import pallas as pl
from jax.experimental.pallas import tpu as pltpu

D = 512
N = 512


def _mlp_head_kernel(x_ref, w1_ref, b1_ref, w2_ref, b2_ref, out_ref, ft_ref):
    @pl.when(pl.program_id(0) == 0)
    def _():
        a = jax.lax.dot_general(
            x_ref[...], w1_ref[...],
            dimension_numbers=(((0,), (1,)), ((), ())),
            preferred_element_type=jnp.float32,
        )
        ft_ref[...] = jnp.maximum(a + b1_ref[...], 0.0)

    acc = jax.lax.dot_general(
        ft_ref[...], w2_ref[...],
        dimension_numbers=(((1,), (1,)), ((), ())),
        preferred_element_type=jnp.float32,
    )
    out_ref[...] = acc + b2_ref[...]


def _mlp_head(x, w1, b1, w2, b2, block_o):
    out_dim = w2.shape[0]
    grid = pl.cdiv(out_dim, block_o)
    return pl.pallas_call(
        _mlp_head_kernel,
        grid=(grid,),
        in_specs=[
            pl.BlockSpec((D, N), lambda i: (0, 0)),
            pl.BlockSpec((D, D), lambda i: (0, 0)),
            pl.BlockSpec((1, D), lambda i: (0, 0)),
            pl.BlockSpec((block_o, D), lambda i: (i, 0)),
            pl.BlockSpec((1, block_o), lambda i: (0, i)),
        ],
        out_specs=pl.BlockSpec((N, block_o), lambda i: (0, i)),
        out_shape=jax.ShapeDtypeStruct((N, out_dim), jnp.float32),
        scratch_shapes=[pltpu.VMEM((N, D), jnp.float32)],
    )(x, w1, b1, w2, b2)


@functools.partial(jax.jit, static_argnames=())
def kernel(seed_features, W1r, b1r, W2r, b2r, W1w, b1w, W2w, b2w):
    Bsz, _, num_seed = seed_features.shape
    x = seed_features.reshape(D, N)
    rot = _mlp_head(x, W1r, b1r.reshape(1, D), W2r, b2r.reshape(1, -1),
                    block_o=2048)
    wid = _mlp_head(x, W1w, b1w.reshape(1, D), W2w, b2w.reshape(1, -1),
                    block_o=2048)
    rotation_scores = rot.reshape(Bsz, num_seed, -1, 5)
    width_pred = wid.reshape(Bsz, num_seed, -1)
    return rotation_scores, width_pred


# speedup vs baseline: 1.0012x; 1.0012x over previous
import functools

import jax
import jax.numpy as jnp
from jax.experimental import pallas as pl
from jax.experimental.pallas import tpu as pltpu

D = 512
N = 512


def _mlp_head_kernel(x_ref, w1_ref, b1_ref, w2_ref, b2_ref, out_ref, ft_ref):
    @pl.when(pl.program_id(0) == 0)
    def _():
        a = jax.lax.dot_general(
            x_ref[...], w1_ref[...],
            dimension_numbers=(((0,), (1,)), ((), ())),
            preferred_element_type=jnp.float32,
        )
        ft_ref[...] = jnp.maximum(a + b1_ref[...], 0.0).astype(jnp.bfloat16)

    acc = jax.lax.dot_general(
        ft_ref[...], w2_ref[...].astype(jnp.bfloat16),
        dimension_numbers=(((1,), (1,)), ((), ())),
        preferred_element_type=jnp.float32,
    )
    out_ref[...] = acc + b2_ref[...]


def _mlp_head(x, w1, b1, w2, b2, block_o):
    out_dim = w2.shape[0]
    grid = pl.cdiv(out_dim, block_o)
    return pl.pallas_call(
        _mlp_head_kernel,
        grid=(grid,),
        in_specs=[
            pl.BlockSpec((D, N), lambda i: (0, 0)),
            pl.BlockSpec((D, D), lambda i: (0, 0)),
            pl.BlockSpec((1, D), lambda i: (0, 0)),
            pl.BlockSpec((block_o, D), lambda i: (i, 0)),
            pl.BlockSpec((1, block_o), lambda i: (0, i)),
        ],
        out_specs=pl.BlockSpec((N, block_o), lambda i: (0, i)),
        out_shape=jax.ShapeDtypeStruct((N, out_dim), jnp.float32),
        scratch_shapes=[pltpu.VMEM((N, D), jnp.bfloat16)],
    )(x, w1, b1, w2, b2)


@functools.partial(jax.jit, static_argnames=())
def kernel(seed_features, W1r, b1r, W2r, b2r, W1w, b1w, W2w, b2w):
    Bsz, _, num_seed = seed_features.shape
    x = seed_features.reshape(D, N)
    rot = _mlp_head(x, W1r, b1r.reshape(1, D), W2r, b2r.reshape(1, -1),
                    block_o=2048)
    wid = _mlp_head(x, W1w, b1w.reshape(1, D), W2w, b2w.reshape(1, -1),
                    block_o=2048)
    rotation_scores = rot.reshape(Bsz, num_seed, -1, 5)
    width_pred = wid.reshape(Bsz, num_seed, -1)
    return rotation_scores, width_pred


# speedup vs baseline: 2.3937x; 2.3909x over previous
import functools

import numpy as np

import jax
import jax.numpy as jnp
from jax.experimental import pallas as pl
from jax.experimental.pallas import tpu as pltpu

D = 512
N = 512
NBINS = 5
BV = 128
BLOCK_O = BV * NBINS


def _layer1(x_ref, w1_ref, b1_ref, ft_ref):
    a = jax.lax.dot_general(
        x_ref[...], w1_ref[...],
        dimension_numbers=(((0,), (1,)), ((), ())),
        preferred_element_type=jnp.float32,
    )
    ft_ref[...] = jnp.maximum(a + b1_ref[...], 0.0).astype(jnp.bfloat16)


def _rot_kernel(rot_out, x_ref, w1_ref, b1_ref, w2_ref, b2_ref, out_ref,
                ft_ref, p_ref):
    @pl.when(pl.program_id(0) == 0)
    def _():
        _layer1(x_ref, w1_ref, b1_ref, ft_ref)
        r = jax.lax.broadcasted_iota(jnp.int32, (BLOCK_O, BLOCK_O), 0)
        c = jax.lax.broadcasted_iota(jnp.int32, (BLOCK_O, BLOCK_O), 1)
        src = NBINS * (r % BV) + r // BV
        p_ref[...] = (c == src).astype(jnp.bfloat16)

    base = pl.program_id(0) * BLOCK_O
    row = jax.lax.broadcasted_iota(jnp.int32, (BLOCK_O, D), 0)
    w2 = jnp.where(base + row < rot_out, w2_ref[...], 0.0)
    w2p = jax.lax.dot_general(
        p_ref[...], w2.astype(jnp.bfloat16),
        dimension_numbers=(((1,), (0,)), ((), ())),
        preferred_element_type=jnp.float32,
    ).astype(jnp.bfloat16)
    acc = jax.lax.dot_general(
        ft_ref[...], w2p,
        dimension_numbers=(((1,), (1,)), ((), ())),
        preferred_element_type=jnp.float32,
    )
    acc = acc + b2_ref[...]
    for b in range(NBINS):
        out_ref[:, b, :] = acc[:, b * BV:(b + 1) * BV]


def _wid_kernel(x_ref, w1_ref, b1_ref, w2_ref, b2_ref, out_ref, ft_ref):
    @pl.when(pl.program_id(0) == 0)
    def _():
        _layer1(x_ref, w1_ref, b1_ref, ft_ref)

    acc = jax.lax.dot_general(
        ft_ref[...], w2_ref[...].astype(jnp.bfloat16),
        dimension_numbers=(((1,), (1,)), ((), ())),
        preferred_element_type=jnp.float32,
    )
    out_ref[...] = acc + b2_ref[...]


def _rot_head(x, w1, b1, w2, b2p):
    n_v = w2.shape[0] // NBINS
    grid = pl.cdiv(n_v, BV)
    return pl.pallas_call(
        functools.partial(_rot_kernel, w2.shape[0]),
        grid=(grid,),
        in_specs=[
            pl.BlockSpec((D, N), lambda i: (0, 0)),
            pl.BlockSpec((D, D), lambda i: (0, 0)),
            pl.BlockSpec((1, D), lambda i: (0, 0)),
            pl.BlockSpec((BLOCK_O, D), lambda i: (i, 0)),
            pl.BlockSpec((1, BLOCK_O), lambda i: (0, i)),
        ],
        out_specs=pl.BlockSpec((N, NBINS, BV), lambda i: (0, 0, i)),
        out_shape=jax.ShapeDtypeStruct((N, NBINS, n_v), jnp.float32),
        scratch_shapes=[pltpu.VMEM((N, D), jnp.bfloat16),
                        pltpu.VMEM((BLOCK_O, BLOCK_O), jnp.bfloat16)],
    )(x, w1, b1, w2, b2p)


def _wid_head(x, w1, b1, w2, b2, block_o):
    out_dim = w2.shape[0]
    grid = pl.cdiv(out_dim, block_o)
    return pl.pallas_call(
        _wid_kernel,
        grid=(grid,),
        in_specs=[
            pl.BlockSpec((D, N), lambda i: (0, 0)),
            pl.BlockSpec((D, D), lambda i: (0, 0)),
            pl.BlockSpec((1, D), lambda i: (0, 0)),
            pl.BlockSpec((block_o, D), lambda i: (i, 0)),
            pl.BlockSpec((1, block_o), lambda i: (0, i)),
        ],
        out_specs=pl.BlockSpec((N, block_o), lambda i: (0, i)),
        out_shape=jax.ShapeDtypeStruct((N, out_dim), jnp.float32),
        scratch_shapes=[pltpu.VMEM((N, D), jnp.bfloat16)],
    )(x, w1, b1, w2, b2)


def _rot_bias_perm_idx(rot_out: int) -> np.ndarray:
    g = np.arange(((rot_out + BLOCK_O - 1) // BLOCK_O) * BLOCK_O)
    k, r = g // BLOCK_O, g % BLOCK_O
    j, dv = r // BV, r % BV
    src = NBINS * (k * BV + dv) + j
    return np.where(src < rot_out, src, 0).astype(np.int32)


@functools.partial(jax.jit, static_argnames=())
def kernel(seed_features, W1r, b1r, W2r, b2r, W1w, b1w, W2w, b2w):
    Bsz, _, num_seed = seed_features.shape
    x = seed_features.reshape(D, N)
    b2rp = b2r[_rot_bias_perm_idx(W2r.shape[0])].reshape(1, -1)
    rot = _rot_head(x, W1r, b1r.reshape(1, D), W2r, b2rp)
    wid = _wid_head(x, W1w, b1w.reshape(1, D), W2w, b2w.reshape(1, -1),
                    block_o=2048)
    rotation_scores = jnp.transpose(rot, (0, 2, 1))[None]
    width_pred = wid.reshape(Bsz, num_seed, -1)
    return rotation_scores, width_pred


# speedup vs baseline: 4.1521x; 1.7346x over previous
import functools

import numpy as np

import jax
import jax.numpy as jnp
from jax.experimental import pallas as pl
from jax.experimental.pallas import tpu as pltpu

D = 512
N = 512
NBINS = 5
BV = 128
BLOCK_O = BV * NBINS


def _layer1(x_ref, w1_ref, b1_ref, ft_ref):
    a = jax.lax.dot_general(
        x_ref[...], w1_ref[...],
        dimension_numbers=(((0,), (1,)), ((), ())),
        preferred_element_type=jnp.float32,
    )
    ft_ref[...] = jnp.maximum(a + b1_ref[...], 0.0).astype(jnp.bfloat16)


def _rot_kernel(rot_out, x_ref, w1_ref, b1_ref, w2_ref, b2_ref, out_ref,
                ft_ref, p_ref):
    @pl.when(pl.program_id(0) == 0)
    def _():
        _layer1(x_ref, w1_ref, b1_ref, ft_ref)
        r = jax.lax.broadcasted_iota(jnp.int32, (BLOCK_O, BLOCK_O), 0)
        c = jax.lax.broadcasted_iota(jnp.int32, (BLOCK_O, BLOCK_O), 1)
        src = NBINS * (r % BV) + r // BV
        p_ref[...] = (c == src).astype(jnp.bfloat16)

    base = pl.program_id(0) * BLOCK_O
    row = jax.lax.broadcasted_iota(jnp.int32, (BLOCK_O, D), 0)
    w2 = jnp.where(base + row < rot_out, w2_ref[...], 0.0)
    w2p = jax.lax.dot_general(
        p_ref[...], w2.astype(jnp.bfloat16),
        dimension_numbers=(((1,), (0,)), ((), ())),
        preferred_element_type=jnp.float32,
    ).astype(jnp.bfloat16)
    acc = jax.lax.dot_general(
        w2p, ft_ref[...],
        dimension_numbers=(((1,), (1,)), ((), ())),
        preferred_element_type=jnp.float32,
    )
    acc = acc + jnp.transpose(b2_ref[...])
    out_ref[...] = acc.reshape(NBINS, BV, N)


def _wid_kernel(x_ref, w1_ref, b1_ref, w2_ref, b2_ref, out_ref, ft_ref):
    @pl.when(pl.program_id(0) == 0)
    def _():
        _layer1(x_ref, w1_ref, b1_ref, ft_ref)

    acc = jax.lax.dot_general(
        w2_ref[...].astype(jnp.bfloat16), ft_ref[...],
        dimension_numbers=(((1,), (1,)), ((), ())),
        preferred_element_type=jnp.float32,
    )
    out_ref[...] = acc + jnp.transpose(b2_ref[...])


def _rot_head(x, w1, b1, w2, b2p):
    n_v = w2.shape[0] // NBINS
    grid = pl.cdiv(n_v, BV)
    return pl.pallas_call(
        functools.partial(_rot_kernel, w2.shape[0]),
        grid=(grid,),
        in_specs=[
            pl.BlockSpec((D, N), lambda i: (0, 0)),
            pl.BlockSpec((D, D), lambda i: (0, 0)),
            pl.BlockSpec((1, D), lambda i: (0, 0)),
            pl.BlockSpec((BLOCK_O, D), lambda i: (i, 0)),
            pl.BlockSpec((1, BLOCK_O), lambda i: (0, i)),
        ],
        out_specs=pl.BlockSpec((NBINS, BV, N), lambda i: (0, i, 0)),
        out_shape=jax.ShapeDtypeStruct((NBINS, n_v, N), jnp.float32),
        scratch_shapes=[pltpu.VMEM((N, D), jnp.bfloat16),
                        pltpu.VMEM((BLOCK_O, BLOCK_O), jnp.bfloat16)],
    )(x, w1, b1, w2, b2p)


def _wid_head(x, w1, b1, w2, b2, block_o):
    out_dim = w2.shape[0]
    grid = pl.cdiv(out_dim, block_o)
    return pl.pallas_call(
        _wid_kernel,
        grid=(grid,),
        in_specs=[
            pl.BlockSpec((D, N), lambda i: (0, 0)),
            pl.BlockSpec((D, D), lambda i: (0, 0)),
            pl.BlockSpec((1, D), lambda i: (0, 0)),
            pl.BlockSpec((block_o, D), lambda i: (i, 0)),
            pl.BlockSpec((1, block_o), lambda i: (0, i)),
        ],
        out_specs=pl.BlockSpec((block_o, N), lambda i: (i, 0)),
        out_shape=jax.ShapeDtypeStruct((out_dim, N), jnp.float32),
        scratch_shapes=[pltpu.VMEM((N, D), jnp.bfloat16)],
    )(x, w1, b1, w2, b2)


def _rot_bias_perm_idx(rot_out: int) -> np.ndarray:
    g = np.arange(((rot_out + BLOCK_O - 1) // BLOCK_O) * BLOCK_O)
    k, r = g // BLOCK_O, g % BLOCK_O
    j, dv = r // BV, r % BV
    src = NBINS * (k * BV + dv) + j
    return np.where(src < rot_out, src, 0).astype(np.int32)


@functools.partial(jax.jit, static_argnames=())
def kernel(seed_features, W1r, b1r, W2r, b2r, W1w, b1w, W2w, b2w):
    Bsz, _, num_seed = seed_features.shape
    x = seed_features.reshape(D, N)
    b2rp = b2r[_rot_bias_perm_idx(W2r.shape[0])].reshape(1, -1)
    rot = _rot_head(x, W1r, b1r.reshape(1, D), W2r, b2rp)
    wid = _wid_head(x, W1w, b1w.reshape(1, D), W2w, b2w.reshape(1, -1),
                    block_o=2048)
    rotation_scores = jnp.transpose(rot, (2, 1, 0))[None]
    width_pred = jnp.transpose(wid)[None]
    return rotation_scores, width_pred
